# baseline (speedup 1.0000x reference)
import dataclasses

import jax
import jax.numpy as jnp
from jax import lax
from jax.experimental import pallas as pl
from jax.experimental.pallas import tpu as pltpu
from jax.experimental.pallas import tpu_sc as plsc

_NC = 2
_NS = 16
_NW = _NC * _NS
_L = 16

_H = 32
_W = 32
_B = 256


def _router_body(e_hbm, t_hbm, out_hbm, e_v, o_v, t_v, sem, sem_out):
    h = lax.axis_index("s") * _NC + lax.axis_index("c")
    cp = pltpu.async_copy(e_hbm.at[h], e_v, sem)
    pltpu.sync_copy(t_hbm, t_v)
    tv = t_v[...]
    ones = jnp.full((_L,), 1, jnp.int32)
    zeros = jnp.zeros((_L,), jnp.int32)
    cp.wait()

    def gate_row(w):
        for b in range(0, _B, _L):
            e = e_v[w, pl.ds(b, _L)]
            coarse = jnp.where(e <= tv, ones, zeros)
            o_v[w, 0, pl.ds(b, _L)] = coarse
            o_v[w, 1, pl.ds(b, _L)] = ones - coarse

    half = _W // 2
    pl.loop(0, half)(gate_row)
    cp_lo = pltpu.async_copy(
        o_v.at[pl.ds(0, half)], out_hbm.at[h, pl.ds(0, half)], sem_out
    )
    pl.loop(half, _W)(gate_row)
    cp_hi = pltpu.async_copy(
        o_v.at[pl.ds(half, half)], out_hbm.at[h, pl.ds(half, half)], sem_out
    )
    cp_lo.wait()
    cp_hi.wait()


def kernel(entropy, threshold):
    e_t = jnp.transpose(entropy, (1, 2, 0))
    tvec = jnp.full((_L,), threshold, jnp.float32)
    mesh = plsc.VectorSubcoreMesh(core_axis_name="c", subcore_axis_name="s")
    cp = pltpu.CompilerParams()
    if "needs_layout_passes" in pltpu.CompilerParams.__dataclass_fields__:
        cp = dataclasses.replace(cp, needs_layout_passes=False)
    run = pl.kernel(
        _router_body,
        out_type=jax.ShapeDtypeStruct((_H, _W, 2, _B), jnp.int32),
        mesh=mesh,
        scratch_types=[
            pltpu.VMEM((_W, _B), jnp.float32),
            pltpu.VMEM((_W, 2, _B), jnp.int32),
            pltpu.VMEM((_L,), jnp.float32),
            pltpu.SemaphoreType.DMA,
            pltpu.SemaphoreType.DMA,
        ],
        compiler_params=cp,
    )
    out = run(e_t, tvec)
    return jnp.transpose(out, (3, 0, 1, 2))


# speedup vs baseline: 1.0421x; 1.0421x over previous
import dataclasses

import jax
import jax.numpy as jnp
from jax import lax
from jax.experimental import pallas as pl
from jax.experimental.pallas import tpu as pltpu
from jax.experimental.pallas import tpu_sc as plsc

_NC = 2
_NS = 16
_NW = _NC * _NS
_L = 16

_H = 32
_W = 32
_B = 256


def _router_body(e_hbm, t_hbm, out_hbm, e_v, o_v, t_v, sem, sem_out):
    h = lax.axis_index("s") * _NC + lax.axis_index("c")
    cp = pltpu.async_copy(e_hbm.at[h], e_v, sem)
    pltpu.sync_copy(t_hbm, t_v)
    tv = t_v[...]
    ones = jnp.full((_L,), 1, jnp.int32)
    zeros = jnp.zeros((_L,), jnp.int32)
    cp.wait()

    def gate_row(w):
        @pl.loop(0, _B, step=_L)
        def _(b):
            e = e_v[w, pl.ds(b, _L)]
            coarse = jnp.where(e <= tv, ones, zeros)
            o_v[w, 0, pl.ds(b, _L)] = coarse
            o_v[w, 1, pl.ds(b, _L)] = ones - coarse

    half = _W // 2
    pl.loop(0, half)(gate_row)
    cp_lo = pltpu.async_copy(
        o_v.at[pl.ds(0, half)], out_hbm.at[h, pl.ds(0, half)], sem_out
    )
    pl.loop(half, _W)(gate_row)
    cp_hi = pltpu.async_copy(
        o_v.at[pl.ds(half, half)], out_hbm.at[h, pl.ds(half, half)], sem_out
    )
    cp_lo.wait()
    cp_hi.wait()


def kernel(entropy, threshold):
    e_t = jnp.transpose(entropy, (1, 2, 0))
    tvec = jnp.full((_L,), threshold, jnp.float32)
    mesh = plsc.VectorSubcoreMesh(core_axis_name="c", subcore_axis_name="s")
    cp = pltpu.CompilerParams()
    if "needs_layout_passes" in pltpu.CompilerParams.__dataclass_fields__:
        cp = dataclasses.replace(cp, needs_layout_passes=False)
    run = pl.kernel(
        _router_body,
        out_type=jax.ShapeDtypeStruct((_H, _W, 2, _B), jnp.int32),
        mesh=mesh,
        scratch_types=[
            pltpu.VMEM((_W, _B), jnp.float32),
            pltpu.VMEM((_W, 2, _B), jnp.int32),
            pltpu.VMEM((_L,), jnp.float32),
            pltpu.SemaphoreType.DMA,
            pltpu.SemaphoreType.DMA,
        ],
        compiler_params=cp,
    )
    out = run(e_t, tvec)
    return jnp.transpose(out, (3, 0, 1, 2))
